# Initial kernel scaffold; baseline (speedup 1.0000x reference)
import jax
import jax.numpy as jnp
from jax.experimental import pallas as pl


def kernel(users, pos_items, neg_items, loss_per_user, w_0, user_embed, item_embed, adj_rows, adj_cols, adj_vals, W, b, noise, drop_mask):
    raise NotImplementedError("write your pallas kernel here")


# speedup vs baseline: 2.7880x; 2.7880x over previous
import functools

import jax
import jax.numpy as jnp
from jax import lax
from jax.experimental import pallas as pl
from jax.experimental.pallas import tpu as pltpu, tpu_sc as plsc

_NU = 50000
_NI = 50000
_D = 64
_NTOT = _NU + _NI
_B = 4096

_NNZ_P = 1 << 20
_NW = 32
_EPW = _NNZ_P // _NW
_CH = 128
_NCHUNK = _EPW // _CH
_SLOTS = 2 * _B
_TRASH = _SLOTS
_COMPACT = _SLOTS + 512
_ZPW = _COMPACT // 16


def _sc_agg_fn():
    mesh = plsc.VectorSubcoreMesh(core_axis_name="c", subcore_axis_name="s")

    @functools.partial(
        pl.kernel,
        mesh=mesh,
        compiler_params=pltpu.CompilerParams(use_tc_tiling_on_sc=False),
        out_type=jax.ShapeDtypeStruct((2, _SLOTS, _D), jnp.float32),
        scratch_types=[
            pltpu.VMEM((_CH,), jnp.int32),
            pltpu.VMEM((_CH,), jnp.int32),
            pltpu.VMEM((_CH,), jnp.int32),
            pltpu.VMEM((_CH,), jnp.int32),
            pltpu.VMEM((_CH,), jnp.float32),
            pltpu.VMEM((_CH,), jnp.float32),
            pltpu.VMEM((_CH, _D), jnp.float32),
            pltpu.VMEM_SHARED((_COMPACT, _D), jnp.float32),
            pltpu.SemaphoreType.DMA,
            pltpu.SemaphoreType.DMA,
        ],
    )
    def sc_agg(table_h, mark_h, rows_h, cols_h, vals_h, drop_h, out_h,
               ridx, cidx, slots, tgt, valsv, dropv, rowsbuf, compact,
               sem1, sem2):
        c = lax.axis_index("c")
        s = lax.axis_index("s")
        wid = s * 2 + c
        zv = jnp.zeros((16,), jnp.float32)
        for j in range(_CH):
            for k in range(4):
                rowsbuf[j, pl.ds(k * 16, 16)] = zv
        for i in range(4):
            pltpu.sync_copy(rowsbuf, compact.at[pl.ds(s * _ZPW + i * _CH, _CH)])
        pltpu.sync_copy(rowsbuf.at[pl.ds(0, _ZPW - 4 * _CH)],
                        compact.at[pl.ds(s * _ZPW + 4 * _CH, _ZPW - 4 * _CH)])
        plsc.subcore_barrier()

        base0 = wid * _EPW

        def chunk(g, carry):
            base = base0 + g * _CH
            pltpu.sync_copy(rows_h.at[pl.ds(base, _CH)], ridx)
            pltpu.sync_copy(cols_h.at[pl.ds(base, _CH)], cidx)
            pltpu.sync_copy(vals_h.at[pl.ds(base, _CH)], valsv)
            pltpu.sync_copy(drop_h.at[pl.ds(base, _CH)], dropv)
            cp1 = pltpu.async_copy(mark_h.at[ridx], slots, sem1)
            cp2 = pltpu.async_copy(table_h.at[cidx], rowsbuf, sem2)
            cp1.wait()
            cp2.wait()
            for t in range(_CH // 16):
                sl = slots[pl.ds(t * 16, 16)]
                w = valsv[pl.ds(t * 16, 16)] * dropv[pl.ds(t * 16, 16)] * 2.0
                tgt[pl.ds(t * 16, 16)] = jnp.where(sl < 0, _TRASH, sl)
                for li in range(16):
                    w_s = w[li]
                    j = t * 16 + li
                    for k in range(4):
                        rowsbuf[j, pl.ds(k * 16, 16)] = (
                            rowsbuf[j, pl.ds(k * 16, 16)] * w_s)
            pltpu.sync_copy(rowsbuf, compact.at[tgt], add=True)
            return carry

        lax.fori_loop(0, _NCHUNK, chunk, 0)
        plsc.subcore_barrier()
        rpw = _SLOTS // 16
        pltpu.sync_copy(compact.at[pl.ds(s * rpw, rpw)],
                        out_h.at[c, pl.ds(s * rpw, rpw)])

    return sc_agg


def _prep_body(emb_ref, noise_ref, out_ref):
    x = emb_ref[...]
    n = noise_ref[...]
    nn = jnp.sqrt(jnp.sum(n * n, axis=1, keepdims=True))
    out_ref[...] = x + jnp.sign(x) * (n / jnp.maximum(nn, 1e-12)) * 0.1


def _lin_body(u_ref, i_ref, w_ref, b_ref, ut0, ut1, it0, it1,
              u1_ref, i1_ref, ut_ref, it_ref):
    wmat = w_ref[...]
    bb = b_ref[...]
    dn = (((1,), (1,)), ((), ()))
    u1_ref[...] = lax.dot_general(u_ref[...], wmat, dn,
                                  preferred_element_type=jnp.float32) + bb
    i1_ref[...] = lax.dot_general(i_ref[...], wmat, dn,
                                  preferred_element_type=jnp.float32) + bb
    ut_ref[...] = ut0[...] + ut1[...]
    it_ref[...] = it0[...] + it1[...]


def _loss_body(pu_ref, zu_ref, pi_ref, zi_ref, out_ref):
    def half(p, z):
        num = jnp.sum(p * z, axis=-1)
        den = jnp.maximum(
            jnp.sqrt(jnp.sum(p * p, axis=-1)) * jnp.sqrt(jnp.sum(z * z, axis=-1)),
            1e-8)
        return -jnp.mean(num / den)

    lu = half(pu_ref[...], zu_ref[...])
    li = half(pi_ref[...], zi_ref[...])
    out_ref[...] = jnp.full((8, 128), 0.5 * lu + 0.5 * li, jnp.float32)


def kernel(users, pos_items, neg_items, loss_per_user, w_0,
           user_embed, item_embed, adj_rows, adj_cols, adj_vals,
           W, b, noise, drop_mask):
    users = users.astype(jnp.int32)
    pos = pos_items.astype(jnp.int32)
    rows = adj_rows.astype(jnp.int32)
    cols = adj_cols.astype(jnp.int32)
    nnz = rows.shape[0]

    all_emb_s = jnp.concatenate([user_embed, item_embed], axis=0)
    all_emb_s = jnp.reshape(all_emb_s, (_D, _NTOT)).T

    blk = 2000
    table = pl.pallas_call(
        _prep_body,
        grid=(_NTOT // blk,),
        in_specs=[pl.BlockSpec((blk, _D), lambda i: (i, 0)),
                  pl.BlockSpec((blk, _D), lambda i: (i, 0))],
        out_specs=pl.BlockSpec((blk, _D), lambda i: (i, 0)),
        out_shape=jax.ShapeDtypeStruct((_NTOT, _D), jnp.float32),
    )(all_emb_s, noise)

    mark = jnp.full((_NTOT,), -1, jnp.int32)
    mark = mark.at[users].set(jnp.arange(_B, dtype=jnp.int32))
    mark = mark.at[_NU + pos].set(jnp.arange(_B, 2 * _B, dtype=jnp.int32))

    pad = _NNZ_P - nnz
    rows_p = jnp.pad(rows, (0, pad))
    cols_p = jnp.pad(cols, (0, pad))
    vals_p = jnp.pad(adj_vals.astype(jnp.float32), (0, pad))
    drop_p = jnp.pad(drop_mask.astype(jnp.float32), (0, pad))

    compact2 = _sc_agg_fn()(table, mark, rows_p, cols_p, vals_p, drop_p)

    slots_u = mark[users]
    slots_i = mark[_NU + pos]
    ut0 = jnp.take(compact2[0], slots_u, axis=0)
    ut1 = jnp.take(compact2[1], slots_u, axis=0)
    it0 = jnp.take(compact2[0], slots_i, axis=0)
    it1 = jnp.take(compact2[1], slots_i, axis=0)
    u_online = jnp.take(user_embed, users, axis=0)
    i_online = jnp.take(item_embed, pos, axis=0)

    fs = jax.ShapeDtypeStruct
    u1, i1, u_t, i_t = pl.pallas_call(
        _lin_body,
        out_shape=[fs((_B, _D), jnp.float32)] * 4,
    )(u_online, i_online, W, b.reshape(1, _D), ut0, ut1, it0, it1)

    u1s = jnp.reshape(u1, (_D, _B)).T
    i1s = jnp.reshape(i1, (_D, _B)).T

    lossmat = pl.pallas_call(
        _loss_body,
        out_shape=fs((8, 128), jnp.float32),
    )(u1s, i_t, i1s, u_t)
    return lossmat[0, 0]
